# Initial kernel scaffold; baseline (speedup 1.0000x reference)
import jax
import jax.numpy as jnp
from jax.experimental import pallas as pl


def kernel(theta_log, seed_prob, Ic, Fc):
    raise NotImplementedError("write your pallas kernel here")


# speedup vs baseline: 68.2741x; 68.2741x over previous
import functools

import jax
import jax.numpy as jnp
from jax.experimental import pallas as pl
from jax.experimental.pallas import tpu as pltpu

_M, _NC, _K = 1024, 512, 32
_HI = jax.lax.Precision.HIGHEST


def _rank_desc_col(x_col, x_row, n):
    gt = (x_row > x_col).astype(jnp.float32)
    i_idx = jax.lax.broadcasted_iota(jnp.int32, (n, n), 0)
    j_idx = jax.lax.broadcasted_iota(jnp.int32, (n, n), 1)
    tie = ((x_row == x_col) & (j_idx < i_idx)).astype(jnp.float32)
    return jnp.sum(gt + tie, axis=1, keepdims=True)


def _rank_desc_row(y_col, y_row, n):
    gt = (y_col > y_row).astype(jnp.float32)
    k_idx = jax.lax.broadcasted_iota(jnp.int32, (n, n), 0)
    j_idx = jax.lax.broadcasted_iota(jnp.int32, (n, n), 1)
    tie = ((y_col == y_row) & (k_idx < j_idx)).astype(jnp.float32)
    return jnp.sum(gt + tie, axis=0, keepdims=True)


def _sorted_pair_loss(x_col, x_row, y_col, y_row, n):
    rx_col = _rank_desc_col(x_col, x_row, n)
    ry_row = _rank_desc_row(y_col, y_row, n)
    match = (rx_col == ry_row)
    cross = jnp.sum(jnp.where(match, x_col * y_row, 0.0))
    sq = jnp.sum(x_col * x_col) + jnp.sum(y_col * y_col)
    return (sq - 2.0 * cross) / n


def _main_body(theta_t_ref, seed_ref, ic_ref, fc_ref, out_ref):
    theta = jnp.log(jax.nn.sigmoid(theta_t_ref[...]))
    t0 = theta[0:1, :]
    t1 = theta[1:2, :]
    t2 = theta[2:3, :]
    c0 = jnp.sum(t0)
    u = t1 - t0
    v = t0 - 2.0 * t1 + t2

    fc = fc_ref[...]
    dimn = (((1,), (1,)), ((), ()))
    s_col = jax.lax.dot_general(fc, u, dimn, precision=_HI)
    s_row = jax.lax.dot_general(u, fc, dimn, precision=_HI)
    g = jax.lax.dot_general(fc * v, fc, dimn, precision=_HI)
    p = c0 + s_col + s_row + g
    i_idx = jax.lax.broadcasted_iota(jnp.int32, (_NC, _NC), 0)
    j_idx = jax.lax.broadcasted_iota(jnp.int32, (_NC, _NC), 1)
    p = jnp.where(i_idx == j_idx, 0.0, p)
    e = jnp.exp(p)

    sp = seed_ref[...]
    sp_max = jnp.max(sp)
    es = jnp.exp(sp - sp_max)
    seed_row = es / jnp.sum(es)
    mask = (ic_ref[...] == 1).astype(jnp.float32)
    group_sum = jax.lax.dot_general(mask, seed_row, dimn, precision=_HI)
    w = mask * (seed_row / group_sum)

    s_mat = jax.lax.dot_general(w, e, (((1,), (0,)), ((), ())),
                                precision=_HI)

    log_s = jnp.log(jnp.where(mask > 0, s_mat, 1.0))
    others = jnp.log1p(-jnp.where(mask > 0, 0.0, s_mat))
    loss = -jnp.sum(log_s) - jnp.sum(others)

    ones_m = jnp.ones((1, _M), dtype=jnp.float32)
    ones_nc = jnp.ones((1, _NC), dtype=jnp.float32)
    d_x_row = jax.lax.dot_general(ones_m, s_mat, (((1,), (0,)), ((), ())),
                                  precision=_HI)
    d_x_col = jax.lax.dot_general(s_mat, ones_m, (((0,), (1,)), ((), ())),
                                  precision=_HI)
    d_y_row = jax.lax.dot_general(ones_m, mask, (((1,), (0,)), ((), ())),
                                  precision=_HI)
    d_y_col = jax.lax.dot_general(mask, ones_m, (((0,), (1,)), ((), ())),
                                  precision=_HI)
    s_x_col = jax.lax.dot_general(s_mat, ones_nc, dimn, precision=_HI)
    s_x_row = jax.lax.dot_general(ones_nc, s_mat, dimn, precision=_HI)
    s_y_col = jax.lax.dot_general(mask, ones_nc, dimn, precision=_HI)
    s_y_row = jax.lax.dot_general(ones_nc, mask, dimn, precision=_HI)

    degree_loss = _sorted_pair_loss(d_x_col, d_x_row, d_y_col, d_y_row, _NC)
    size_loss = _sorted_pair_loss(s_x_col, s_x_row, s_y_col, s_y_row, _M)

    out_ref[0, 0] = loss + degree_loss + size_loss


@jax.jit
def _run(theta_log, seed_prob, Ic, Fc):
    theta_t = theta_log.T
    seed2 = seed_prob.reshape(1, _NC)
    fc_f = Fc.astype(jnp.float32)
    out = pl.pallas_call(
        _main_body,
        out_shape=jax.ShapeDtypeStruct((1, 1), jnp.float32),
        out_specs=pl.BlockSpec(memory_space=pltpu.SMEM),
    )(theta_t, seed2, Ic, fc_f)
    return out[0, 0]


def kernel(theta_log, seed_prob, Ic, Fc):
    return _run(theta_log, seed_prob, Ic, Fc)
